# baseline (speedup 1.0000x reference)
import functools

import jax
import jax.numpy as jnp
from jax import lax
from jax.experimental import pallas as pl
from jax.experimental.pallas import tpu as pltpu
from jax.experimental.pallas import tpu_sc as plsc

NU = 25000
NI = 25000
N = NU + NI
D = 64
TD = 80

BLK1 = 1000


def _stage1_body(x_ref, w_ref, y_ref, pm_ref):
    x = x_ref[...]
    y_ref[...] = jnp.dot(x, w_ref[...], preferred_element_type=jnp.float32)
    pm_ref[...] = jnp.max(jnp.sum(x * x, axis=1)).reshape(1, 1, 1)


def _stage1(x_t, W_cg):
    nb = N // BLK1
    return pl.pallas_call(
        _stage1_body,
        grid=(nb,),
        in_specs=[
            pl.BlockSpec((BLK1, D), lambda i: (i, 0)),
            pl.BlockSpec((D, D), lambda i: (0, 0)),
        ],
        out_specs=[
            pl.BlockSpec((BLK1, D), lambda i: (i, 0)),
            pl.BlockSpec((1, 1, 1), lambda i: (i, 0, 0)),
        ],
        out_shape=[
            jax.ShapeDtypeStruct((N, D), jnp.float32),
            jax.ShapeDtypeStruct((nb, 1, 1), jnp.float32),
        ],
    )(x_t, W_cg)


def _stage3_body(x_ref, emb_ref, msg_ref, wt_ref, wlin_ref, n2_ref, ts_ref,
                 enc_ref, tbl_ref, lin_ref):
    inv = lax.rsqrt(n2_ref[0, 0])
    e = jnp.exp(-ts_ref[0, 0])
    x = x_ref[...]
    emb = emb_ref[...]
    z = (x + e * jax.nn.relu(msg_ref[...] + emb)) * inv
    enc_ref[...] = jnp.concatenate([z, emb], axis=1)
    t = jnp.dot(z, wt_ref[...].T, preferred_element_type=jnp.float32)
    blk = t.shape[0]
    ones = jnp.ones((blk, 1), jnp.float32)
    zeros = jnp.zeros((blk, TD - D - 1), jnp.float32)
    tbl_ref[...] = jnp.concatenate([t, ones, zeros], axis=1)
    lin_ref[...] = jnp.dot(z, wlin_ref[...].T, preferred_element_type=jnp.float32)


def _stage3(x_half, emb_half, msg, msg_row0, W_t, W_lin, norm2, tsd):
    nb = NU // BLK1
    off = msg_row0 // BLK1
    return pl.pallas_call(
        _stage3_body,
        grid=(nb,),
        in_specs=[
            pl.BlockSpec((BLK1, D), lambda i: (i, 0)),
            pl.BlockSpec((BLK1, D), lambda i: (i, 0)),
            pl.BlockSpec((BLK1, D), lambda i, o=off: (i + o, 0)),
            pl.BlockSpec((D, D), lambda i: (0, 0)),
            pl.BlockSpec((D, D), lambda i: (0, 0)),
            pl.BlockSpec((1, 1), lambda i: (0, 0)),
            pl.BlockSpec((1, 1), lambda i: (0, 0)),
        ],
        out_specs=[
            pl.BlockSpec((BLK1, 2 * D), lambda i: (i, 0)),
            pl.BlockSpec((BLK1, TD), lambda i: (i, 0)),
            pl.BlockSpec((BLK1, D), lambda i: (i, 0)),
        ],
        out_shape=[
            jax.ShapeDtypeStruct((NU, 2 * D), jnp.float32),
            jax.ShapeDtypeStruct((NU, TD), jnp.float32),
            jax.ShapeDtypeStruct((NU, D), jnp.float32),
        ],
    )(x_half, emb_half, msg, W_t, W_lin, norm2, tsd)


def _stage5_body(lin_ref, agg_ref, enc_ref, b_ref, tp_ref, p_ref):
    z = enc_ref[:, :D]
    d = jax.nn.relu(lin_ref[...] + b_ref[...] + agg_ref[:, :D])
    m = (agg_ref[:, D:D + 1] > 0).astype(jnp.float32)
    d = d * m
    tp_ref[...] = z + d
    p_ref[...] = jnp.stack([jnp.sum(d * d), jnp.sum(m)]).reshape(1, 1, 2)


def _stage5(lin, agg, enc, b):
    nb = NU // BLK1
    return pl.pallas_call(
        _stage5_body,
        grid=(nb,),
        in_specs=[
            pl.BlockSpec((BLK1, D), lambda i: (i, 0)),
            pl.BlockSpec((BLK1, TD), lambda i: (i, 0)),
            pl.BlockSpec((BLK1, 2 * D), lambda i: (i, 0)),
            pl.BlockSpec((1, D), lambda i: (0, 0)),
        ],
        out_specs=[
            pl.BlockSpec((BLK1, D), lambda i: (i, 0)),
            pl.BlockSpec((1, 1, 2), lambda i: (i, 0, 0)),
        ],
        out_shape=[
            jax.ShapeDtypeStruct((NU, D), jnp.float32),
            jax.ShapeDtypeStruct((nb, 1, 2), jnp.float32),
        ],
    )(lin, agg, enc, b.reshape(1, D))


def _pred_body(xu_ref, xi_ref, wu_ref, bu_ref, wi_ref, bi_ref, pu_ref, pi_ref,
               lr_ref, lj_ref):
    B = 1024
    NNEG = 5
    hu = jnp.dot(xu_ref[...], wu_ref[...].T, preferred_element_type=jnp.float32) + bu_ref[...]
    hi = jnp.dot(xi_ref[...], wi_ref[...].T, preferred_element_type=jnp.float32) + bi_ref[...]
    hu_pos = hu[:B]
    hi_pos = hi[:B]
    hu_neg = hu[B:].reshape(B, NNEG, 2 * D)
    hi_neg = hi[B:].reshape(B, NNEG, 2 * D)
    pos = jnp.sum(hu_pos * hi_pos, axis=1, keepdims=True)
    neg_u = jnp.sum(hu_pos[:, None, :] * hi_neg, axis=2)
    neg_i = jnp.sum(hu_neg * hi_pos[:, None, :], axis=2)
    scores = jnp.concatenate([pos, neg_u, neg_i], axis=1)
    mx = jnp.max(scores, axis=1, keepdims=True)
    lse = jnp.log(jnp.sum(jnp.exp(scores - mx), axis=1, keepdims=True))
    ls0 = scores[:, :1] - mx - lse
    lr_ref[...] = (-jnp.mean(ls0)).reshape(1, 1)
    pu = pu_ref[...]
    pi = pi_ref[...]
    lj_ref[...] = (jnp.sum(pu[..., 0]) / jnp.sum(pu[..., 1])
                   + jnp.sum(pi[..., 0]) / jnp.sum(pi[..., 1])).reshape(1, 1)


def _predictor(xu_tgt, xi_tgt, W_up, b_up, W_ip, b_ip, part_u, part_i):
    return pl.pallas_call(
        _pred_body,
        out_shape=[
            jax.ShapeDtypeStruct((1, 1), jnp.float32),
            jax.ShapeDtypeStruct((1, 1), jnp.float32),
        ],
    )(xu_tgt, xi_tgt, W_up, b_up.reshape(1, 2 * D), W_ip, b_ip.reshape(1, 2 * D),
      part_u, part_i)


NS = 16
ZR = 48
EO = 800000
EI = 100000
EIP = 100352
NTGT = 6144
KT = 96

_MESH = plsc.VectorSubcoreMesh(core_axis_name="c", subcore_axis_name="s")


def _make_agg_kernel(width, half, k, m, nb, with_tgt):
    assert k % 16 == 0 and m % 2 == 0 and nb % 2 == 0
    ib = m * k
    eps = ib * nb
    ntrash = 1024
    acc_r = -(-(half + ntrash) // ZR) * ZR
    copies = acc_r // ZR
    zt = -(-copies // NS)
    sp = (-(-half // NS) + 7) // 8 * 8
    sp_last = half - (NS - 1) * sp
    assert 0 < sp_last <= sp and acc_r >= half + ntrash

    out_type = [jax.ShapeDtypeStruct((2 * half, width), jnp.float32)]
    scratch = [
        pltpu.VMEM((ib,), jnp.int32), pltpu.VMEM((ib,), jnp.int32),
        pltpu.VMEM((ib,), jnp.int32), pltpu.VMEM((ib,), jnp.int32),
        pltpu.VMEM((m, k), jnp.int32), pltpu.VMEM((m, k), jnp.int32),
        pltpu.VMEM((k, width), jnp.float32), pltpu.VMEM((k, width), jnp.float32),
        pltpu.VMEM((ZR, width), jnp.float32),
        pltpu.VMEM_SHARED((acc_r, width), jnp.float32),
        pltpu.SemaphoreType.DMA, pltpu.SemaphoreType.DMA,
        pltpu.SemaphoreType.DMA, pltpu.SemaphoreType.DMA,
        pltpu.SemaphoreType.DMA,
    ]
    if with_tgt:
        out_type.append(jax.ShapeDtypeStruct((NTGT, 2 * D), jnp.float32))
        scratch += [pltpu.VMEM((KT,), jnp.int32),
                    pltpu.VMEM((KT, 2 * D), jnp.float32)]

    def body(*refs):
        if with_tgt:
            (tbl_hbm, g_hbm, s_hbm, enc_hbm, tidx_hbm, out_hbm, tout_hbm,
             g0, g1, d0, d1, dl0, dl1, r0, r1, zbuf, acc,
             ib0, ib1, gs0, gs1, zs, ti_v, tr_v) = refs
        else:
            (tbl_hbm, g_hbm, s_hbm, out_hbm,
             g0, g1, d0, d1, dl0, dl1, r0, r1, zbuf, acc,
             ib0, ib1, gs0, gs1, zs) = refs
        gv, dv, dlv, rows = (g0, g1), (d0, d1), (dl0, dl1), (r0, r1)
        ibs, gss = (ib0, ib1), (gs0, gs1)
        c = lax.axis_index("c")
        s = lax.axis_index("s")
        lo = c * half
        iota16 = lax.iota(jnp.int32, 16)

        def fire_iblock(bbv, par):
            base = s * eps + bbv * ib
            pltpu.async_copy(g_hbm.at[pl.ds(base, ib)], gv[par], ibs[par])
            pltpu.async_copy(s_hbm.at[pl.ds(base, ib)], dv[par], ibs[par])

        def wait_iblock(par):
            pltpu.make_async_copy(g_hbm.at[pl.ds(0, ib)], gv[par], ibs[par]).wait()
            pltpu.make_async_copy(s_hbm.at[pl.ds(0, ib)], dv[par], ibs[par]).wait()

        def transform(par):
            for j in range(m):
                for i in range(k // 16):
                    d = dv[par][pl.ds(j * k + 16 * i, 16)]
                    mm = (d >= lo) & (d < lo + half)
                    toff = half + (16 * (j * (k // 16) + i)) % ntrash
                    dlv[par][j, pl.ds(16 * i, 16)] = jnp.where(
                        mm, d - lo, toff + iota16)

        def fire_g(par, ipar, j):
            pltpu.async_copy(tbl_hbm.at[gv[ipar].at[pl.ds(j * k, k)]],
                             rows[par], gss[par])

        def wait_g(par):
            pltpu.make_async_copy(tbl_hbm.at[gv[0].at[pl.ds(0, k)]],
                                  rows[par], gss[par]).wait()

        def sync_s(par, ipar, j):
            pltpu.sync_copy(rows[par], acc.at[dlv[ipar].at[j]], add=True)

        fire_iblock(0, 0)
        fire_iblock(1, 1)
        z16 = jnp.zeros((16,), jnp.float32)
        for r in range(ZR):
            for j in range(width // 16):
                zbuf[r, pl.ds(16 * j, 16)] = z16

        def zfire(t, carry):
            kk = s + NS * t

            @pl.when(kk < copies)
            def _():
                pltpu.async_copy(zbuf, acc.at[pl.ds(kk * ZR, ZR)], zs)
            return carry

        def zdrain(t, carry):
            kk = s + NS * t

            @pl.when(kk < copies)
            def _():
                pltpu.make_async_copy(zbuf, acc.at[pl.ds(0, ZR)], zs).wait()
            return carry

        lax.fori_loop(0, zt, zfire, None)
        lax.fori_loop(0, zt, zdrain, None)
        plsc.subcore_barrier()
        wait_iblock(0)
        transform(0)
        fire_g(0, 0, 0)

        def block_pair(bb2, carry):
            for b in (0, 1):
                bb = 2 * bb2 + b
                for j in range(m):
                    p, pn = j % 2, (j + 1) % 2
                    wait_g(p)
                    if j < m - 1:
                        fire_g(pn, b, j + 1)
                    sync_s(p, b, j)

                @pl.when(bb < nb - 1)
                def _(b=b):
                    wait_iblock(1 - b)
                    transform(1 - b)
                    fire_g(0, 1 - b, 0)

                @pl.when(bb < nb - 2)
                def _(b=b, bb=bb):
                    fire_iblock(bb + 2, b)
            return carry

        lax.fori_loop(0, nb // 2, block_pair, None)

        if with_tgt:
            w = s * 2 + c
            for q in range(NTGT // (2 * NS * KT)):
                tb = w * (NTGT // (2 * NS)) + q * KT
                pltpu.sync_copy(tidx_hbm.at[pl.ds(tb, KT)], ti_v)
                pltpu.async_copy(enc_hbm.at[ti_v], tr_v, gs0).wait()
                pltpu.sync_copy(tr_v, tout_hbm.at[pl.ds(tb, KT)])

        plsc.subcore_barrier()

        @pl.when(s < NS - 1)
        def _():
            pltpu.sync_copy(acc.at[pl.ds(s * sp, sp)],
                            out_hbm.at[pl.ds(c * half + s * sp, sp)])

        @pl.when(s == NS - 1)
        def _():
            pltpu.sync_copy(acc.at[pl.ds((NS - 1) * sp, sp_last)],
                            out_hbm.at[pl.ds(c * half + (NS - 1) * sp, sp_last)])

    return functools.partial(
        pl.kernel, mesh=_MESH,
        compiler_params=pltpu.CompilerParams(use_tc_tiling_on_sc=False),
        out_type=out_type if with_tgt else out_type[0],
        scratch_types=scratch,
    )(body)


EOP = 819200

_obs_kernel = _make_agg_kernel(D, N // 2, 128, 8, 50, False)
_ins_kernel = _make_agg_kernel(TD, NU // 2, 112, 14, 4, True)


def _obs_segsum(y, obs_src, obs_dst):
    npad = EOP - EO
    pad_g = jnp.arange(npad, dtype=jnp.int32) % N
    pad_s = jnp.full((npad,), N, jnp.int32)
    return _obs_kernel(y, jnp.concatenate([obs_src, pad_g]),
                       jnp.concatenate([obs_dst, pad_s]))


def _sparse_stage4(t_iu, t_ui, ins_u, ins_i, tgtu_all, tgti_all, xu_enc, xi_enc):
    npad = EIP - EI
    pad_g = (jnp.arange(npad, dtype=jnp.int32) % NU)
    pad_s = jnp.full((npad,), NU, jnp.int32)
    g0 = jnp.concatenate([ins_i, pad_g])
    s0 = jnp.concatenate([ins_u, pad_s])
    g1 = jnp.concatenate([ins_u, pad_g])
    s1 = jnp.concatenate([ins_i, pad_s])
    agg_iu, xu_tgt = _ins_kernel(t_iu, g0, s0, xu_enc, tgtu_all)
    agg_ui, xi_tgt = _ins_kernel(t_ui, g1, s1, xi_enc, tgti_all)
    return agg_iu, agg_ui, xu_tgt, xi_tgt


def kernel(ts_diff, obs_src, obs_dst, ins_u, ins_i, tgt_u, tgt_i, tgt_u_neg,
           tgt_i_neg, xu_in, xi_in, embeds_u, embeds_i, W_cg, W_uu, b_uu, W_ii,
           b_ii, W_ui, W_iu, W_up, b_up, W_ip, b_ip):
    x_t = jnp.concatenate([xu_in, xi_in], axis=0)
    y, pmax = _stage1(x_t, W_cg)
    norm2 = jnp.max(pmax).reshape(1, 1)
    tsd = ts_diff.reshape(1, 1)

    msg = _obs_segsum(y, obs_src, obs_dst)

    xu_enc, t_ui, xu_lin = _stage3(xu_in, embeds_u, msg, 0, W_ui, W_uu, norm2, tsd)
    xi_enc, t_iu, xi_lin = _stage3(xi_in, embeds_i, msg, NU, W_iu, W_ii, norm2, tsd)

    tgtu_all = jnp.concatenate([tgt_u.reshape(-1), tgt_u_neg.reshape(-1)])
    tgti_all = jnp.concatenate([tgt_i.reshape(-1), tgt_i_neg.reshape(-1)])
    agg_iu, agg_ui, xu_tgt, xi_tgt = _sparse_stage4(
        t_iu, t_ui, ins_u, ins_i, tgtu_all, tgti_all, xu_enc, xi_enc)

    xu_tp, part_u = _stage5(xu_lin, agg_iu, xu_enc, b_uu)
    xi_tp, part_i = _stage5(xi_lin, agg_ui, xi_enc, b_ii)

    lr, lj = _predictor(xu_tgt, xi_tgt, W_up, b_up, W_ip, b_ip, part_u, part_i)

    B = tgt_u.shape[0]
    loss_rec = lr.reshape(())
    loss_jump = lj.reshape(())
    xu_pos = xu_tgt[:B].reshape(B, 1, 2 * D)
    xi_enc_out = xi_enc.reshape(NI, 1, 2 * D)
    return (loss_rec, loss_jump, xu_tp, xi_tp, xu_pos, xi_enc_out)


# speedup vs baseline: 1.0411x; 1.0411x over previous
import functools

import jax
import jax.numpy as jnp
from jax import lax
from jax.experimental import pallas as pl
from jax.experimental.pallas import tpu as pltpu
from jax.experimental.pallas import tpu_sc as plsc

NU = 25000
NI = 25000
N = NU + NI
D = 64
TD = 80

BLK1 = 1000


def _stage1_body(x_ref, w_ref, y_ref, pm_ref):
    x = x_ref[...]
    y_ref[...] = jnp.dot(x, w_ref[...], preferred_element_type=jnp.float32)
    pm_ref[...] = jnp.max(jnp.sum(x * x, axis=1)).reshape(1, 1, 1)


def _stage1(x_t, W_cg):
    nb = N // BLK1
    return pl.pallas_call(
        _stage1_body,
        grid=(nb,),
        in_specs=[
            pl.BlockSpec((BLK1, D), lambda i: (i, 0)),
            pl.BlockSpec((D, D), lambda i: (0, 0)),
        ],
        out_specs=[
            pl.BlockSpec((BLK1, D), lambda i: (i, 0)),
            pl.BlockSpec((1, 1, 1), lambda i: (i, 0, 0)),
        ],
        out_shape=[
            jax.ShapeDtypeStruct((N, D), jnp.float32),
            jax.ShapeDtypeStruct((nb, 1, 1), jnp.float32),
        ],
    )(x_t, W_cg)


def _stage3_body(x_ref, emb_ref, msg_ref, wt_ref, wlin_ref, n2_ref, ts_ref,
                 enc_ref, tbl_ref, lin_ref):
    inv = lax.rsqrt(n2_ref[0, 0])
    e = jnp.exp(-ts_ref[0, 0])
    x = x_ref[...]
    emb = emb_ref[...]
    z = (x + e * jax.nn.relu(msg_ref[...] + emb)) * inv
    enc_ref[...] = jnp.concatenate([z, emb], axis=1)
    t = jnp.dot(z, wt_ref[...].T, preferred_element_type=jnp.float32)
    blk = t.shape[0]
    ones = jnp.ones((blk, 1), jnp.float32)
    zeros = jnp.zeros((blk, TD - D - 1), jnp.float32)
    tbl_ref[...] = jnp.concatenate([t, ones, zeros], axis=1)
    lin_ref[...] = jnp.dot(z, wlin_ref[...].T, preferred_element_type=jnp.float32)


def _stage3(x_half, emb_half, msg, msg_row0, W_t, W_lin, norm2, tsd):
    nb = NU // BLK1
    off = msg_row0 // BLK1
    return pl.pallas_call(
        _stage3_body,
        grid=(nb,),
        in_specs=[
            pl.BlockSpec((BLK1, D), lambda i: (i, 0)),
            pl.BlockSpec((BLK1, D), lambda i: (i, 0)),
            pl.BlockSpec((BLK1, D), lambda i, o=off: (i + o, 0)),
            pl.BlockSpec((D, D), lambda i: (0, 0)),
            pl.BlockSpec((D, D), lambda i: (0, 0)),
            pl.BlockSpec((1, 1), lambda i: (0, 0)),
            pl.BlockSpec((1, 1), lambda i: (0, 0)),
        ],
        out_specs=[
            pl.BlockSpec((BLK1, 2 * D), lambda i: (i, 0)),
            pl.BlockSpec((BLK1, TD), lambda i: (i, 0)),
            pl.BlockSpec((BLK1, D), lambda i: (i, 0)),
        ],
        out_shape=[
            jax.ShapeDtypeStruct((NU, 2 * D), jnp.float32),
            jax.ShapeDtypeStruct((NU, TD), jnp.float32),
            jax.ShapeDtypeStruct((NU, D), jnp.float32),
        ],
    )(x_half, emb_half, msg, W_t, W_lin, norm2, tsd)


def _stage5_body(lin_ref, agg_ref, enc_ref, b_ref, tp_ref, p_ref):
    z = enc_ref[:, :D]
    d = jax.nn.relu(lin_ref[...] + b_ref[...] + agg_ref[:, :D])
    m = (agg_ref[:, D:D + 1] > 0).astype(jnp.float32)
    d = d * m
    tp_ref[...] = z + d
    p_ref[...] = jnp.stack([jnp.sum(d * d), jnp.sum(m)]).reshape(1, 1, 2)


def _stage5(lin, agg, enc, b):
    nb = NU // BLK1
    return pl.pallas_call(
        _stage5_body,
        grid=(nb,),
        in_specs=[
            pl.BlockSpec((BLK1, D), lambda i: (i, 0)),
            pl.BlockSpec((BLK1, TD), lambda i: (i, 0)),
            pl.BlockSpec((BLK1, 2 * D), lambda i: (i, 0)),
            pl.BlockSpec((1, D), lambda i: (0, 0)),
        ],
        out_specs=[
            pl.BlockSpec((BLK1, D), lambda i: (i, 0)),
            pl.BlockSpec((1, 1, 2), lambda i: (i, 0, 0)),
        ],
        out_shape=[
            jax.ShapeDtypeStruct((NU, D), jnp.float32),
            jax.ShapeDtypeStruct((nb, 1, 2), jnp.float32),
        ],
    )(lin, agg, enc, b.reshape(1, D))


def _pred_body(xu_ref, xi_ref, wu_ref, bu_ref, wi_ref, bi_ref, pu_ref, pi_ref,
               lr_ref, lj_ref):
    B = 1024
    NNEG = 5
    hu = jnp.dot(xu_ref[...], wu_ref[...].T, preferred_element_type=jnp.float32) + bu_ref[...]
    hi = jnp.dot(xi_ref[...], wi_ref[...].T, preferred_element_type=jnp.float32) + bi_ref[...]
    hu_pos = hu[:B]
    hi_pos = hi[:B]
    hu_neg = hu[B:].reshape(B, NNEG, 2 * D)
    hi_neg = hi[B:].reshape(B, NNEG, 2 * D)
    pos = jnp.sum(hu_pos * hi_pos, axis=1, keepdims=True)
    neg_u = jnp.sum(hu_pos[:, None, :] * hi_neg, axis=2)
    neg_i = jnp.sum(hu_neg * hi_pos[:, None, :], axis=2)
    scores = jnp.concatenate([pos, neg_u, neg_i], axis=1)
    mx = jnp.max(scores, axis=1, keepdims=True)
    lse = jnp.log(jnp.sum(jnp.exp(scores - mx), axis=1, keepdims=True))
    ls0 = scores[:, :1] - mx - lse
    lr_ref[...] = (-jnp.mean(ls0)).reshape(1, 1)
    pu = pu_ref[...]
    pi = pi_ref[...]
    lj_ref[...] = (jnp.sum(pu[..., 0]) / jnp.sum(pu[..., 1])
                   + jnp.sum(pi[..., 0]) / jnp.sum(pi[..., 1])).reshape(1, 1)


def _predictor(xu_tgt, xi_tgt, W_up, b_up, W_ip, b_ip, part_u, part_i):
    return pl.pallas_call(
        _pred_body,
        out_shape=[
            jax.ShapeDtypeStruct((1, 1), jnp.float32),
            jax.ShapeDtypeStruct((1, 1), jnp.float32),
        ],
    )(xu_tgt, xi_tgt, W_up, b_up.reshape(1, 2 * D), W_ip, b_ip.reshape(1, 2 * D),
      part_u, part_i)


NS = 16
ZR = 32
EO = 800000
EI = 100000
EIP = 100352
NTGT = 6144
KT = 96

_MESH = plsc.VectorSubcoreMesh(core_axis_name="c", subcore_axis_name="s")


def _make_agg_kernel(width, half, k, m, nb, with_tgt):
    assert k % 16 == 0 and m % 2 == 0 and nb % 2 == 0 and (m * k) % (2 * k) == 0
    ib = m * k
    eps = ib * nb
    ntrash = 256
    acc_r = -(-(half + ntrash) // ZR) * ZR
    copies = acc_r // ZR
    zt = -(-copies // NS)
    sp = (-(-half // NS) + 7) // 8 * 8
    sp_last = half - (NS - 1) * sp
    assert 0 < sp_last <= sp and acc_r >= half + ntrash

    out_type = [jax.ShapeDtypeStruct((2 * half, width), jnp.float32)]
    scratch = [
        pltpu.VMEM((ib,), jnp.int32), pltpu.VMEM((ib,), jnp.int32),
        pltpu.VMEM((ib,), jnp.int32), pltpu.VMEM((ib,), jnp.int32),
        pltpu.VMEM((ib + 16,), jnp.int32), pltpu.VMEM((ib + 16,), jnp.int32),
        pltpu.VMEM((ib + 16,), jnp.int32), pltpu.VMEM((ib + 16,), jnp.int32),
        pltpu.VMEM((k,), jnp.int32), pltpu.VMEM((k,), jnp.int32),
        pltpu.VMEM((16,), jnp.int32),
        pltpu.SMEM((16,), jnp.int32),
        pltpu.VMEM((k, width), jnp.float32), pltpu.VMEM((k, width), jnp.float32),
        pltpu.VMEM((ZR, width), jnp.float32),
        pltpu.VMEM_SHARED((acc_r, width), jnp.float32),
        pltpu.SemaphoreType.DMA, pltpu.SemaphoreType.DMA,
        pltpu.SemaphoreType.DMA, pltpu.SemaphoreType.DMA,
        pltpu.SemaphoreType.DMA,
    ]
    if with_tgt:
        out_type.append(jax.ShapeDtypeStruct((NTGT, 2 * D), jnp.float32))
        scratch += [pltpu.VMEM((KT,), jnp.int32),
                    pltpu.VMEM((KT, 2 * D), jnp.float32)]

    def body(*refs):
        if with_tgt:
            (tbl_hbm, g_hbm, s_hbm, enc_hbm, tidx_hbm, out_hbm, tout_hbm,
             g0, g1, d0, d1, gc0, gc1, dc0, dc1, dk0, dk1, cntb, scnt, r0, r1,
             zbuf, acc, ib0, ib1, gs0, gs1, zs, ti_v, tr_v) = refs
        else:
            (tbl_hbm, g_hbm, s_hbm, out_hbm,
             g0, g1, d0, d1, gc0, gc1, dc0, dc1, dk0, dk1, cntb, scnt, r0, r1,
             zbuf, acc, ib0, ib1, gs0, gs1, zs) = refs
        gv, dv, rows = (g0, g1), (d0, d1), (r0, r1)
        glc, dlc, dlk = (gc0, gc1), (dc0, dc1), (dk0, dk1)
        ibs, gss = (ib0, ib1), (gs0, gs1)
        c = lax.axis_index("c")
        s = lax.axis_index("s")
        lo = c * half
        iota16 = lax.iota(jnp.int32, 16)

        def fire_iblock(bbv, par):
            base = s * eps + bbv * ib
            pltpu.async_copy(g_hbm.at[pl.ds(base, ib)], gv[par], ibs[par])
            pltpu.async_copy(s_hbm.at[pl.ds(base, ib)], dv[par], ibs[par])

        def wait_iblock(par):
            pltpu.make_async_copy(g_hbm.at[pl.ds(0, ib)], gv[par], ibs[par]).wait()
            pltpu.make_async_copy(s_hbm.at[pl.ds(0, ib)], dv[par], ibs[par]).wait()

        def compact(par):
            for i in range(ib // 16):
                glc[par][pl.ds(16 * i, 16)] = (16 * i) % 512 + iota16
                dlc[par][pl.ds(16 * i, 16)] = (half + (16 * i) % ntrash) + iota16
            offv = jnp.zeros((16,), jnp.int32)
            for i in range(ib // 16):
                d = dv[par][pl.ds(16 * i, 16)]
                g = gv[par][pl.ds(16 * i, 16)]
                mm = (d >= lo) & (d < lo + half)
                mmi = mm.astype(jnp.int32)
                cs = mmi
                for st in (1, 2, 4, 8):
                    sh = cs.at[jnp.maximum(iota16 - st, 0)].get(
                        mode="promise_in_bounds")
                    cs = cs + jnp.where(iota16 >= st, sh, 0)
                pos = jnp.where(mm, offv + cs - 1, ib + iota16)
                plsc.store_scatter(glc[par], [pos], g)
                plsc.store_scatter(dlc[par], [pos], d - lo)
                offv = offv + plsc.all_reduce_population_count(mm)
            cntb[...] = offv
            return cntb[...][0]

        def fire_g(par, ipar, cc):
            pltpu.async_copy(tbl_hbm.at[glc[ipar].at[pl.ds(cc * k, k)]],
                             rows[par], gss[par])

        def wait_g(par):
            pltpu.make_async_copy(tbl_hbm.at[glc[0].at[pl.ds(0, k)]],
                                  rows[par], gss[par]).wait()

        def sync_s(par, ipar, cc):
            for i in range(k // 16):
                dlk[par][pl.ds(16 * i, 16)] = dlc[ipar][pl.ds(cc * k + 16 * i, 16)]
            pltpu.sync_copy(rows[par], acc.at[dlk[par]], add=True)

        fire_iblock(0, 0)
        fire_iblock(1, 1)
        z16 = jnp.zeros((16,), jnp.float32)
        for r in range(ZR):
            for j in range(width // 16):
                zbuf[r, pl.ds(16 * j, 16)] = z16

        def zfire(t, carry):
            kk = s + NS * t

            @pl.when(kk < copies)
            def _():
                pltpu.async_copy(zbuf, acc.at[pl.ds(kk * ZR, ZR)], zs)
            return carry

        def zdrain(t, carry):
            kk = s + NS * t

            @pl.when(kk < copies)
            def _():
                pltpu.make_async_copy(zbuf, acc.at[pl.ds(0, ZR)], zs).wait()
            return carry

        lax.fori_loop(0, zt, zfire, None)
        lax.fori_loop(0, zt, zdrain, None)
        plsc.subcore_barrier()
        wait_iblock(0)
        n0 = compact(0)

        def block_pair(bb2, n_cur):
            for b in (0, 1):
                bb = 2 * bb2 + b
                nch2 = (n_cur + 2 * k - 1) // (2 * k)

                @pl.when(nch2 > 0)
                def _(b=b):
                    fire_g(0, b, 0)

                def inner(t, carry, b=b, nch2=nch2):
                    for b2 in (0, 1):
                        cc = 2 * t + b2
                        wait_g(b2)

                        @pl.when(cc + 1 < 2 * nch2)
                        def _(b=b, b2=b2, cc=cc):
                            fire_g(1 - b2, b, cc + 1)
                        sync_s(b2, b, cc)
                    return carry

                lax.fori_loop(0, nch2, inner, None)
                wait_iblock(1 - b)
                n_cur = compact(1 - b)
                fire_iblock(jnp.minimum(bb + 2, nb - 1), b)
            return n_cur

        lax.fori_loop(0, nb // 2, block_pair, n0)

        if with_tgt:
            w = s * 2 + c
            for q in range(NTGT // (2 * NS * KT)):
                tb = w * (NTGT // (2 * NS)) + q * KT
                pltpu.sync_copy(tidx_hbm.at[pl.ds(tb, KT)], ti_v)
                pltpu.async_copy(enc_hbm.at[ti_v], tr_v, gs0).wait()
                pltpu.sync_copy(tr_v, tout_hbm.at[pl.ds(tb, KT)])

        plsc.subcore_barrier()

        @pl.when(s < NS - 1)
        def _():
            pltpu.sync_copy(acc.at[pl.ds(s * sp, sp)],
                            out_hbm.at[pl.ds(c * half + s * sp, sp)])

        @pl.when(s == NS - 1)
        def _():
            pltpu.sync_copy(acc.at[pl.ds((NS - 1) * sp, sp_last)],
                            out_hbm.at[pl.ds(c * half + (NS - 1) * sp, sp_last)])

    return functools.partial(
        pl.kernel, mesh=_MESH,
        compiler_params=pltpu.CompilerParams(use_tc_tiling_on_sc=False, needs_layout_passes=False),
        out_type=out_type if with_tgt else out_type[0],
        scratch_types=scratch,
    )(body)


EOP = 819200

_obs_kernel = _make_agg_kernel(D, N // 2, 128, 8, 50, False)
_ins_kernel = _make_agg_kernel(TD, NU // 2, 112, 14, 4, True)


def _obs_segsum(y, obs_src, obs_dst):
    npad = EOP - EO
    pad_g = jnp.arange(npad, dtype=jnp.int32) % N
    pad_s = jnp.full((npad,), N, jnp.int32)
    return _obs_kernel(y, jnp.concatenate([obs_src, pad_g]),
                       jnp.concatenate([obs_dst, pad_s]))


def _sparse_stage4(t_iu, t_ui, ins_u, ins_i, tgtu_all, tgti_all, xu_enc, xi_enc):
    npad = EIP - EI
    pad_g = (jnp.arange(npad, dtype=jnp.int32) % NU)
    pad_s = jnp.full((npad,), NU, jnp.int32)
    g0 = jnp.concatenate([ins_i, pad_g])
    s0 = jnp.concatenate([ins_u, pad_s])
    g1 = jnp.concatenate([ins_u, pad_g])
    s1 = jnp.concatenate([ins_i, pad_s])
    agg_iu, xu_tgt = _ins_kernel(t_iu, g0, s0, xu_enc, tgtu_all)
    agg_ui, xi_tgt = _ins_kernel(t_ui, g1, s1, xi_enc, tgti_all)
    return agg_iu, agg_ui, xu_tgt, xi_tgt


def kernel(ts_diff, obs_src, obs_dst, ins_u, ins_i, tgt_u, tgt_i, tgt_u_neg,
           tgt_i_neg, xu_in, xi_in, embeds_u, embeds_i, W_cg, W_uu, b_uu, W_ii,
           b_ii, W_ui, W_iu, W_up, b_up, W_ip, b_ip):
    x_t = jnp.concatenate([xu_in, xi_in], axis=0)
    y, pmax = _stage1(x_t, W_cg)
    norm2 = jnp.max(pmax).reshape(1, 1)
    tsd = ts_diff.reshape(1, 1)

    msg = _obs_segsum(y, obs_src, obs_dst)

    xu_enc, t_ui, xu_lin = _stage3(xu_in, embeds_u, msg, 0, W_ui, W_uu, norm2, tsd)
    xi_enc, t_iu, xi_lin = _stage3(xi_in, embeds_i, msg, NU, W_iu, W_ii, norm2, tsd)

    tgtu_all = jnp.concatenate([tgt_u.reshape(-1), tgt_u_neg.reshape(-1)])
    tgti_all = jnp.concatenate([tgt_i.reshape(-1), tgt_i_neg.reshape(-1)])
    agg_iu, agg_ui, xu_tgt, xi_tgt = _sparse_stage4(
        t_iu, t_ui, ins_u, ins_i, tgtu_all, tgti_all, xu_enc, xi_enc)

    xu_tp, part_u = _stage5(xu_lin, agg_iu, xu_enc, b_uu)
    xi_tp, part_i = _stage5(xi_lin, agg_ui, xi_enc, b_ii)

    lr, lj = _predictor(xu_tgt, xi_tgt, W_up, b_up, W_ip, b_ip, part_u, part_i)

    B = tgt_u.shape[0]
    loss_rec = lr.reshape(())
    loss_jump = lj.reshape(())
    xu_pos = xu_tgt[:B].reshape(B, 1, 2 * D)
    xi_enc_out = xi_enc.reshape(NI, 1, 2 * D)
    return (loss_rec, loss_jump, xu_tp, xi_tp, xu_pos, xi_enc_out)


# speedup vs baseline: 1.0972x; 1.0539x over previous
import functools

import jax
import jax.numpy as jnp
from jax import lax
from jax.experimental import pallas as pl
from jax.experimental.pallas import tpu as pltpu
from jax.experimental.pallas import tpu_sc as plsc

NU = 25000
NI = 25000
N = NU + NI
D = 64
TD = 80

BLK1 = 1000


def _stage1_body(x_ref, w_ref, y_ref, pm_ref):
    x = x_ref[...]
    y_ref[...] = jnp.dot(x, w_ref[...], preferred_element_type=jnp.float32)
    pm_ref[...] = jnp.max(jnp.sum(x * x, axis=1)).reshape(1, 1, 1)


def _stage1(x_t, W_cg):
    nb = N // BLK1
    return pl.pallas_call(
        _stage1_body,
        grid=(nb,),
        in_specs=[
            pl.BlockSpec((BLK1, D), lambda i: (i, 0)),
            pl.BlockSpec((D, D), lambda i: (0, 0)),
        ],
        out_specs=[
            pl.BlockSpec((BLK1, D), lambda i: (i, 0)),
            pl.BlockSpec((1, 1, 1), lambda i: (i, 0, 0)),
        ],
        out_shape=[
            jax.ShapeDtypeStruct((N, D), jnp.float32),
            jax.ShapeDtypeStruct((nb, 1, 1), jnp.float32),
        ],
    )(x_t, W_cg)


def _stage3_body(x_ref, emb_ref, msg_ref, wt_ref, wlin_ref, n2_ref, ts_ref,
                 enc_ref, tbl_ref, lin_ref):
    inv = lax.rsqrt(n2_ref[0, 0])
    e = jnp.exp(-ts_ref[0, 0])
    x = x_ref[...]
    emb = emb_ref[...]
    z = (x + e * jax.nn.relu(msg_ref[...] + emb)) * inv
    enc_ref[...] = jnp.concatenate([z, emb], axis=1)
    t = jnp.dot(z, wt_ref[...].T, preferred_element_type=jnp.float32)
    blk = t.shape[0]
    ones = jnp.ones((blk, 1), jnp.float32)
    zeros = jnp.zeros((blk, TD - D - 1), jnp.float32)
    tbl_ref[...] = jnp.concatenate([t, ones, zeros], axis=1)
    lin_ref[...] = jnp.dot(z, wlin_ref[...].T, preferred_element_type=jnp.float32)


def _stage3(x_half, emb_half, msg, msg_row0, W_t, W_lin, norm2, tsd):
    nb = NU // BLK1
    off = msg_row0 // BLK1
    return pl.pallas_call(
        _stage3_body,
        grid=(nb,),
        in_specs=[
            pl.BlockSpec((BLK1, D), lambda i: (i, 0)),
            pl.BlockSpec((BLK1, D), lambda i: (i, 0)),
            pl.BlockSpec((BLK1, D), lambda i, o=off: (i + o, 0)),
            pl.BlockSpec((D, D), lambda i: (0, 0)),
            pl.BlockSpec((D, D), lambda i: (0, 0)),
            pl.BlockSpec((1, 1), lambda i: (0, 0)),
            pl.BlockSpec((1, 1), lambda i: (0, 0)),
        ],
        out_specs=[
            pl.BlockSpec((BLK1, 2 * D), lambda i: (i, 0)),
            pl.BlockSpec((BLK1, TD), lambda i: (i, 0)),
            pl.BlockSpec((BLK1, D), lambda i: (i, 0)),
        ],
        out_shape=[
            jax.ShapeDtypeStruct((NU, 2 * D), jnp.float32),
            jax.ShapeDtypeStruct((NU, TD), jnp.float32),
            jax.ShapeDtypeStruct((NU, D), jnp.float32),
        ],
    )(x_half, emb_half, msg, W_t, W_lin, norm2, tsd)


def _stage5_body(lin_ref, agg_ref, enc_ref, b_ref, tp_ref, p_ref):
    z = enc_ref[:, :D]
    d = jax.nn.relu(lin_ref[...] + b_ref[...] + agg_ref[:, :D])
    m = (agg_ref[:, D:D + 1] > 0).astype(jnp.float32)
    d = d * m
    tp_ref[...] = z + d
    p_ref[...] = jnp.stack([jnp.sum(d * d), jnp.sum(m)]).reshape(1, 1, 2)


def _stage5(lin, agg, enc, b):
    nb = NU // BLK1
    return pl.pallas_call(
        _stage5_body,
        grid=(nb,),
        in_specs=[
            pl.BlockSpec((BLK1, D), lambda i: (i, 0)),
            pl.BlockSpec((BLK1, TD), lambda i: (i, 0)),
            pl.BlockSpec((BLK1, 2 * D), lambda i: (i, 0)),
            pl.BlockSpec((1, D), lambda i: (0, 0)),
        ],
        out_specs=[
            pl.BlockSpec((BLK1, D), lambda i: (i, 0)),
            pl.BlockSpec((1, 1, 2), lambda i: (i, 0, 0)),
        ],
        out_shape=[
            jax.ShapeDtypeStruct((NU, D), jnp.float32),
            jax.ShapeDtypeStruct((nb, 1, 2), jnp.float32),
        ],
    )(lin, agg, enc, b.reshape(1, D))


def _pred_body(xu_ref, xi_ref, wu_ref, bu_ref, wi_ref, bi_ref, pu_ref, pi_ref,
               lr_ref, lj_ref):
    B = 1024
    NNEG = 5
    hu = jnp.dot(xu_ref[...], wu_ref[...].T, preferred_element_type=jnp.float32) + bu_ref[...]
    hi = jnp.dot(xi_ref[...], wi_ref[...].T, preferred_element_type=jnp.float32) + bi_ref[...]
    hu_pos = hu[:B]
    hi_pos = hi[:B]
    hu_neg = hu[B:].reshape(B, NNEG, 2 * D)
    hi_neg = hi[B:].reshape(B, NNEG, 2 * D)
    pos = jnp.sum(hu_pos * hi_pos, axis=1, keepdims=True)
    neg_u = jnp.sum(hu_pos[:, None, :] * hi_neg, axis=2)
    neg_i = jnp.sum(hu_neg * hi_pos[:, None, :], axis=2)
    scores = jnp.concatenate([pos, neg_u, neg_i], axis=1)
    mx = jnp.max(scores, axis=1, keepdims=True)
    lse = jnp.log(jnp.sum(jnp.exp(scores - mx), axis=1, keepdims=True))
    ls0 = scores[:, :1] - mx - lse
    lr_ref[...] = (-jnp.mean(ls0)).reshape(1, 1)
    pu = pu_ref[...]
    pi = pi_ref[...]
    lj_ref[...] = (jnp.sum(pu[..., 0]) / jnp.sum(pu[..., 1])
                   + jnp.sum(pi[..., 0]) / jnp.sum(pi[..., 1])).reshape(1, 1)


def _predictor(xu_tgt, xi_tgt, W_up, b_up, W_ip, b_ip, part_u, part_i):
    return pl.pallas_call(
        _pred_body,
        out_shape=[
            jax.ShapeDtypeStruct((1, 1), jnp.float32),
            jax.ShapeDtypeStruct((1, 1), jnp.float32),
        ],
    )(xu_tgt, xi_tgt, W_up, b_up.reshape(1, 2 * D), W_ip, b_ip.reshape(1, 2 * D),
      part_u, part_i)


NS = 16
ZR = 32
EO = 800000
EI = 100000
EIP = 100352
NTGT = 6144
KT = 96

_MESH = plsc.VectorSubcoreMesh(core_axis_name="c", subcore_axis_name="s")


def _make_agg_kernel(width, half, k, m, nb, with_tgt):
    assert k % 16 == 0 and m % 2 == 0 and nb % 2 == 0 and (m * k) % (2 * k) == 0
    ib = m * k
    eps = ib * nb
    ntrash = 256
    acc_r = -(-(half + ntrash) // ZR) * ZR
    copies = acc_r // ZR
    zt = -(-copies // NS)
    sp = (-(-half // NS) + 7) // 8 * 8
    sp_last = half - (NS - 1) * sp
    assert 0 < sp_last <= sp and acc_r >= half + ntrash

    out_type = [jax.ShapeDtypeStruct((2 * half, width), jnp.float32)]
    scratch = [
        pltpu.VMEM((ib,), jnp.int32), pltpu.VMEM((ib,), jnp.int32),
        pltpu.VMEM((ib,), jnp.int32), pltpu.VMEM((ib,), jnp.int32),
        pltpu.VMEM((ib + 16,), jnp.int32), pltpu.VMEM((ib + 16,), jnp.int32),
        pltpu.VMEM((ib + 16,), jnp.int32), pltpu.VMEM((ib + 16,), jnp.int32),
        pltpu.VMEM((k,), jnp.int32), pltpu.VMEM((k,), jnp.int32),
        pltpu.VMEM((16,), jnp.int32),
        pltpu.SMEM((16,), jnp.int32),
        pltpu.VMEM((k, width), jnp.float32), pltpu.VMEM((k, width), jnp.float32),
        pltpu.VMEM((ZR, width), jnp.float32),
        pltpu.VMEM_SHARED((acc_r, width), jnp.float32),
        pltpu.SemaphoreType.DMA, pltpu.SemaphoreType.DMA,
        pltpu.SemaphoreType.DMA, pltpu.SemaphoreType.DMA,
        pltpu.SemaphoreType.DMA,
    ]
    if with_tgt:
        out_type.append(jax.ShapeDtypeStruct((NTGT, 2 * D), jnp.float32))
        scratch += [pltpu.VMEM((KT,), jnp.int32),
                    pltpu.VMEM((KT, 2 * D), jnp.float32)]

    def body(*refs):
        if with_tgt:
            (tbl_hbm, g_hbm, s_hbm, enc_hbm, tidx_hbm, out_hbm, tout_hbm,
             g0, g1, d0, d1, gc0, gc1, dc0, dc1, dk0, dk1, cntb, scnt, r0, r1,
             zbuf, acc, ib0, ib1, gs0, gs1, zs, ti_v, tr_v) = refs
        else:
            (tbl_hbm, g_hbm, s_hbm, out_hbm,
             g0, g1, d0, d1, gc0, gc1, dc0, dc1, dk0, dk1, cntb, scnt, r0, r1,
             zbuf, acc, ib0, ib1, gs0, gs1, zs) = refs
        gv, dv, rows = (g0, g1), (d0, d1), (r0, r1)
        glc, dlc, dlk = (gc0, gc1), (dc0, dc1), (dk0, dk1)
        ibs, gss = (ib0, ib1), (gs0, gs1)
        c = lax.axis_index("c")
        s = lax.axis_index("s")
        lo = c * half
        iota16 = lax.iota(jnp.int32, 16)

        def fire_iblock(bbv, par):
            base = s * eps + bbv * ib
            pltpu.async_copy(g_hbm.at[pl.ds(base, ib)], gv[par], ibs[par])
            pltpu.async_copy(s_hbm.at[pl.ds(base, ib)], dv[par], ibs[par])

        def wait_iblock(par):
            pltpu.make_async_copy(g_hbm.at[pl.ds(0, ib)], gv[par], ibs[par]).wait()
            pltpu.make_async_copy(s_hbm.at[pl.ds(0, ib)], dv[par], ibs[par]).wait()

        def compact(par):
            for i in range(ib // 16):
                glc[par][pl.ds(16 * i, 16)] = (16 * i) % 512 + iota16
                dlc[par][pl.ds(16 * i, 16)] = (half + (16 * i) % ntrash) + iota16
            offv = jnp.zeros((16,), jnp.int32)
            for i in range(ib // 16):
                d = dv[par][pl.ds(16 * i, 16)]
                g = gv[par][pl.ds(16 * i, 16)]
                mm = (d >= lo) & (d < lo + half)
                mmi = mm.astype(jnp.int32)
                cs = mmi
                for st in (1, 2, 4, 8):
                    sh = cs.at[jnp.maximum(iota16 - st, 0)].get(
                        mode="promise_in_bounds")
                    cs = cs + jnp.where(iota16 >= st, sh, 0)
                pos = jnp.where(mm, offv + cs - 1, ib + iota16)
                plsc.store_scatter(glc[par], [pos], g)
                plsc.store_scatter(dlc[par], [pos], d - lo)
                offv = offv + plsc.all_reduce_population_count(mm)
            cntb[...] = offv
            return cntb[...][0]

        def fire_g(par, ipar, cc):
            pltpu.async_copy(tbl_hbm.at[glc[ipar].at[pl.ds(cc * k, k)]],
                             rows[par], gss[par])

        def wait_g(par):
            pltpu.make_async_copy(tbl_hbm.at[glc[0].at[pl.ds(0, k)]],
                                  rows[par], gss[par]).wait()

        def sync_s(par, ipar, cc):
            for i in range(k // 16):
                dlk[par][pl.ds(16 * i, 16)] = dlc[ipar][pl.ds(cc * k + 16 * i, 16)]
            pltpu.sync_copy(rows[par], acc.at[dlk[par]], add=True)

        fire_iblock(0, 0)
        fire_iblock(1, 1)
        z16 = jnp.zeros((16,), jnp.float32)
        for r in range(ZR):
            for j in range(width // 16):
                zbuf[r, pl.ds(16 * j, 16)] = z16

        def zfire(t, carry):
            kk = s + NS * t

            @pl.when(kk < copies)
            def _():
                pltpu.async_copy(zbuf, acc.at[pl.ds(kk * ZR, ZR)], zs)
            return carry

        def zdrain(t, carry):
            kk = s + NS * t

            @pl.when(kk < copies)
            def _():
                pltpu.make_async_copy(zbuf, acc.at[pl.ds(0, ZR)], zs).wait()
            return carry

        lax.fori_loop(0, zt, zfire, None)
        lax.fori_loop(0, zt, zdrain, None)
        plsc.subcore_barrier()
        wait_iblock(0)
        n0 = compact(0)

        def block_pair(bb2, n_cur):
            for b in (0, 1):
                bb = 2 * bb2 + b
                nch2 = (n_cur + 2 * k - 1) // (2 * k)

                @pl.when(nch2 > 0)
                def _(b=b):
                    fire_g(0, b, 0)

                wait_iblock(1 - b)
                n_cur = compact(1 - b)
                fire_iblock(jnp.minimum(bb + 2, nb - 1), b)

                def inner(t, carry, b=b, nch2=nch2):
                    for b2 in (0, 1):
                        cc = 2 * t + b2
                        wait_g(b2)

                        @pl.when(cc + 1 < 2 * nch2)
                        def _(b=b, b2=b2, cc=cc):
                            fire_g(1 - b2, b, cc + 1)
                        sync_s(b2, b, cc)
                    return carry

                lax.fori_loop(0, nch2, inner, None)
            return n_cur

        lax.fori_loop(0, nb // 2, block_pair, n0)

        if with_tgt:
            w = s * 2 + c
            for q in range(NTGT // (2 * NS * KT)):
                tb = w * (NTGT // (2 * NS)) + q * KT
                pltpu.sync_copy(tidx_hbm.at[pl.ds(tb, KT)], ti_v)
                pltpu.async_copy(enc_hbm.at[ti_v], tr_v, gs0).wait()
                pltpu.sync_copy(tr_v, tout_hbm.at[pl.ds(tb, KT)])

        plsc.subcore_barrier()

        @pl.when(s < NS - 1)
        def _():
            pltpu.sync_copy(acc.at[pl.ds(s * sp, sp)],
                            out_hbm.at[pl.ds(c * half + s * sp, sp)])

        @pl.when(s == NS - 1)
        def _():
            pltpu.sync_copy(acc.at[pl.ds((NS - 1) * sp, sp_last)],
                            out_hbm.at[pl.ds(c * half + (NS - 1) * sp, sp_last)])

    return functools.partial(
        pl.kernel, mesh=_MESH,
        compiler_params=pltpu.CompilerParams(use_tc_tiling_on_sc=False, needs_layout_passes=False),
        out_type=out_type if with_tgt else out_type[0],
        scratch_types=scratch,
    )(body)


EOP = 819200

_obs_kernel = _make_agg_kernel(D, N // 2, 128, 8, 50, False)
_ins_kernel = _make_agg_kernel(TD, NU // 2, 112, 14, 4, True)


def _obs_segsum(y, obs_src, obs_dst):
    npad = EOP - EO
    pad_g = jnp.arange(npad, dtype=jnp.int32) % N
    pad_s = jnp.full((npad,), N, jnp.int32)
    return _obs_kernel(y, jnp.concatenate([obs_src, pad_g]),
                       jnp.concatenate([obs_dst, pad_s]))


def _sparse_stage4(t_iu, t_ui, ins_u, ins_i, tgtu_all, tgti_all, xu_enc, xi_enc):
    npad = EIP - EI
    pad_g = (jnp.arange(npad, dtype=jnp.int32) % NU)
    pad_s = jnp.full((npad,), NU, jnp.int32)
    g0 = jnp.concatenate([ins_i, pad_g])
    s0 = jnp.concatenate([ins_u, pad_s])
    g1 = jnp.concatenate([ins_u, pad_g])
    s1 = jnp.concatenate([ins_i, pad_s])
    agg_iu, xu_tgt = _ins_kernel(t_iu, g0, s0, xu_enc, tgtu_all)
    agg_ui, xi_tgt = _ins_kernel(t_ui, g1, s1, xi_enc, tgti_all)
    return agg_iu, agg_ui, xu_tgt, xi_tgt


def kernel(ts_diff, obs_src, obs_dst, ins_u, ins_i, tgt_u, tgt_i, tgt_u_neg,
           tgt_i_neg, xu_in, xi_in, embeds_u, embeds_i, W_cg, W_uu, b_uu, W_ii,
           b_ii, W_ui, W_iu, W_up, b_up, W_ip, b_ip):
    x_t = jnp.concatenate([xu_in, xi_in], axis=0)
    y, pmax = _stage1(x_t, W_cg)
    norm2 = jnp.max(pmax).reshape(1, 1)
    tsd = ts_diff.reshape(1, 1)

    msg = _obs_segsum(y, obs_src, obs_dst)

    xu_enc, t_ui, xu_lin = _stage3(xu_in, embeds_u, msg, 0, W_ui, W_uu, norm2, tsd)
    xi_enc, t_iu, xi_lin = _stage3(xi_in, embeds_i, msg, NU, W_iu, W_ii, norm2, tsd)

    tgtu_all = jnp.concatenate([tgt_u.reshape(-1), tgt_u_neg.reshape(-1)])
    tgti_all = jnp.concatenate([tgt_i.reshape(-1), tgt_i_neg.reshape(-1)])
    agg_iu, agg_ui, xu_tgt, xi_tgt = _sparse_stage4(
        t_iu, t_ui, ins_u, ins_i, tgtu_all, tgti_all, xu_enc, xi_enc)

    xu_tp, part_u = _stage5(xu_lin, agg_iu, xu_enc, b_uu)
    xi_tp, part_i = _stage5(xi_lin, agg_ui, xi_enc, b_ii)

    lr, lj = _predictor(xu_tgt, xi_tgt, W_up, b_up, W_ip, b_ip, part_u, part_i)

    B = tgt_u.shape[0]
    loss_rec = lr.reshape(())
    loss_jump = lj.reshape(())
    xu_pos = xu_tgt[:B].reshape(B, 1, 2 * D)
    xi_enc_out = xi_enc.reshape(NI, 1, 2 * D)
    return (loss_rec, loss_jump, xu_tp, xi_tp, xu_pos, xi_enc_out)


# speedup vs baseline: 1.1819x; 1.0772x over previous
import functools

import jax
import jax.numpy as jnp
from jax import lax
from jax.experimental import pallas as pl
from jax.experimental.pallas import tpu as pltpu
from jax.experimental.pallas import tpu_sc as plsc

NU = 25000
NI = 25000
N = NU + NI
D = 64
TD = 80

BLK1 = 5000


def _stage1_body(x_ref, w_ref, y_ref, pm_ref):
    x = x_ref[...]
    y_ref[...] = jnp.dot(x, w_ref[...], preferred_element_type=jnp.float32)
    pm_ref[...] = jnp.max(jnp.sum(x * x, axis=1)).reshape(1, 1, 1)


def _stage1(x_t, W_cg):
    nb = N // BLK1
    return pl.pallas_call(
        _stage1_body,
        grid=(nb,),
        in_specs=[
            pl.BlockSpec((BLK1, D), lambda i: (i, 0)),
            pl.BlockSpec((D, D), lambda i: (0, 0)),
        ],
        out_specs=[
            pl.BlockSpec((BLK1, D), lambda i: (i, 0)),
            pl.BlockSpec((1, 1, 1), lambda i: (i, 0, 0)),
        ],
        out_shape=[
            jax.ShapeDtypeStruct((N, D), jnp.float32),
            jax.ShapeDtypeStruct((nb, 1, 1), jnp.float32),
        ],
    )(x_t, W_cg)


def _stage3_body(x_ref, emb_ref, msg_ref, wt_ref, wlin_ref, n2_ref, ts_ref,
                 enc_ref, tbl_ref, lin_ref):
    inv = lax.rsqrt(n2_ref[0, 0])
    e = jnp.exp(-ts_ref[0, 0])
    x = x_ref[...]
    emb = emb_ref[...]
    z = (x + e * jax.nn.relu(msg_ref[...] + emb)) * inv
    enc_ref[...] = jnp.concatenate([z, emb], axis=1)
    t = jnp.dot(z, wt_ref[...].T, preferred_element_type=jnp.float32)
    blk = t.shape[0]
    ones = jnp.ones((blk, 1), jnp.float32)
    zeros = jnp.zeros((blk, TD - D - 1), jnp.float32)
    tbl_ref[...] = jnp.concatenate([t, ones, zeros], axis=1)
    lin_ref[...] = jnp.dot(z, wlin_ref[...].T, preferred_element_type=jnp.float32)


def _stage3(x_half, emb_half, msg, msg_row0, W_t, W_lin, norm2, tsd):
    nb = NU // BLK1
    off = msg_row0 // BLK1
    return pl.pallas_call(
        _stage3_body,
        grid=(nb,),
        in_specs=[
            pl.BlockSpec((BLK1, D), lambda i: (i, 0)),
            pl.BlockSpec((BLK1, D), lambda i: (i, 0)),
            pl.BlockSpec((BLK1, D), lambda i, o=off: (i + o, 0)),
            pl.BlockSpec((D, D), lambda i: (0, 0)),
            pl.BlockSpec((D, D), lambda i: (0, 0)),
            pl.BlockSpec((1, 1), lambda i: (0, 0)),
            pl.BlockSpec((1, 1), lambda i: (0, 0)),
        ],
        out_specs=[
            pl.BlockSpec((BLK1, 2 * D), lambda i: (i, 0)),
            pl.BlockSpec((BLK1, TD), lambda i: (i, 0)),
            pl.BlockSpec((BLK1, D), lambda i: (i, 0)),
        ],
        out_shape=[
            jax.ShapeDtypeStruct((NU, 2 * D), jnp.float32),
            jax.ShapeDtypeStruct((NU, TD), jnp.float32),
            jax.ShapeDtypeStruct((NU, D), jnp.float32),
        ],
    )(x_half, emb_half, msg, W_t, W_lin, norm2, tsd)


def _stage5_body(lin_ref, agg_ref, enc_ref, b_ref, tp_ref, p_ref):
    z = enc_ref[:, :D]
    d = jax.nn.relu(lin_ref[...] + b_ref[...] + agg_ref[:, :D])
    m = (agg_ref[:, D:D + 1] > 0).astype(jnp.float32)
    d = d * m
    tp_ref[...] = z + d
    p_ref[...] = jnp.stack([jnp.sum(d * d), jnp.sum(m)]).reshape(1, 1, 2)


def _stage5(lin, agg, enc, b):
    nb = NU // BLK1
    return pl.pallas_call(
        _stage5_body,
        grid=(nb,),
        in_specs=[
            pl.BlockSpec((BLK1, D), lambda i: (i, 0)),
            pl.BlockSpec((BLK1, TD), lambda i: (i, 0)),
            pl.BlockSpec((BLK1, 2 * D), lambda i: (i, 0)),
            pl.BlockSpec((1, D), lambda i: (0, 0)),
        ],
        out_specs=[
            pl.BlockSpec((BLK1, D), lambda i: (i, 0)),
            pl.BlockSpec((1, 1, 2), lambda i: (i, 0, 0)),
        ],
        out_shape=[
            jax.ShapeDtypeStruct((NU, D), jnp.float32),
            jax.ShapeDtypeStruct((nb, 1, 2), jnp.float32),
        ],
    )(lin, agg, enc, b.reshape(1, D))


def _pred_body(xu_ref, xi_ref, wu_ref, bu_ref, wi_ref, bi_ref, pu_ref, pi_ref,
               lr_ref, lj_ref):
    B = 1024
    NNEG = 5
    hu = jnp.dot(xu_ref[...], wu_ref[...].T, preferred_element_type=jnp.float32) + bu_ref[...]
    hi = jnp.dot(xi_ref[...], wi_ref[...].T, preferred_element_type=jnp.float32) + bi_ref[...]
    hu_pos = hu[:B]
    hi_pos = hi[:B]
    hu_neg = hu[B:].reshape(B, NNEG, 2 * D)
    hi_neg = hi[B:].reshape(B, NNEG, 2 * D)
    pos = jnp.sum(hu_pos * hi_pos, axis=1, keepdims=True)
    neg_u = jnp.sum(hu_pos[:, None, :] * hi_neg, axis=2)
    neg_i = jnp.sum(hu_neg * hi_pos[:, None, :], axis=2)
    scores = jnp.concatenate([pos, neg_u, neg_i], axis=1)
    mx = jnp.max(scores, axis=1, keepdims=True)
    lse = jnp.log(jnp.sum(jnp.exp(scores - mx), axis=1, keepdims=True))
    ls0 = scores[:, :1] - mx - lse
    lr_ref[...] = (-jnp.mean(ls0)).reshape(1, 1)
    pu = pu_ref[...]
    pi = pi_ref[...]
    lj_ref[...] = (jnp.sum(pu[..., 0]) / jnp.sum(pu[..., 1])
                   + jnp.sum(pi[..., 0]) / jnp.sum(pi[..., 1])).reshape(1, 1)


def _predictor(xu_tgt, xi_tgt, W_up, b_up, W_ip, b_ip, part_u, part_i):
    return pl.pallas_call(
        _pred_body,
        out_shape=[
            jax.ShapeDtypeStruct((1, 1), jnp.float32),
            jax.ShapeDtypeStruct((1, 1), jnp.float32),
        ],
    )(xu_tgt, xi_tgt, W_up, b_up.reshape(1, 2 * D), W_ip, b_ip.reshape(1, 2 * D),
      part_u, part_i)


NS = 16
ZR = 32
EO = 800000
EI = 100000
EIP = 100352
NTGT = 6144
KT = 96

_MESH = plsc.VectorSubcoreMesh(core_axis_name="c", subcore_axis_name="s")


def _make_agg_kernel(width, half, k, m, nb, with_tgt):
    assert k % 16 == 0 and m % 2 == 0 and nb % 2 == 0 and (m * k) % (2 * k) == 0
    ib = m * k
    eps = ib * nb
    ntrash = 256
    acc_r = -(-(half + ntrash) // ZR) * ZR
    copies = acc_r // ZR
    zt = -(-copies // NS)
    sp = (-(-half // NS) + 7) // 8 * 8
    sp_last = half - (NS - 1) * sp
    assert 0 < sp_last <= sp and acc_r >= half + ntrash

    out_type = [jax.ShapeDtypeStruct((2 * half, width), jnp.float32)]
    scratch = [
        pltpu.VMEM((ib,), jnp.int32), pltpu.VMEM((ib,), jnp.int32),
        pltpu.VMEM((ib,), jnp.int32), pltpu.VMEM((ib,), jnp.int32),
        pltpu.VMEM((ib + 16,), jnp.int32), pltpu.VMEM((ib + 16,), jnp.int32),
        pltpu.VMEM((ib + 16,), jnp.int32), pltpu.VMEM((ib + 16,), jnp.int32),
        pltpu.VMEM((k,), jnp.int32), pltpu.VMEM((k,), jnp.int32),
        pltpu.VMEM((16,), jnp.int32),
        pltpu.SMEM((16,), jnp.int32),
        pltpu.VMEM((k, width), jnp.float32), pltpu.VMEM((k, width), jnp.float32),
        pltpu.VMEM((ZR, width), jnp.float32),
        pltpu.VMEM_SHARED((acc_r, width), jnp.float32),
        pltpu.SemaphoreType.DMA, pltpu.SemaphoreType.DMA,
        pltpu.SemaphoreType.DMA, pltpu.SemaphoreType.DMA,
        pltpu.SemaphoreType.DMA,
    ]
    if with_tgt:
        out_type.append(jax.ShapeDtypeStruct((NTGT, 2 * D), jnp.float32))
        scratch += [pltpu.VMEM((KT,), jnp.int32),
                    pltpu.VMEM((KT, 2 * D), jnp.float32)]

    def body(*refs):
        if with_tgt:
            (tbl_hbm, g_hbm, s_hbm, enc_hbm, tidx_hbm, out_hbm, tout_hbm,
             g0, g1, d0, d1, gc0, gc1, dc0, dc1, dk0, dk1, cntb, scnt, r0, r1,
             zbuf, acc, ib0, ib1, gs0, gs1, zs, ti_v, tr_v) = refs
        else:
            (tbl_hbm, g_hbm, s_hbm, out_hbm,
             g0, g1, d0, d1, gc0, gc1, dc0, dc1, dk0, dk1, cntb, scnt, r0, r1,
             zbuf, acc, ib0, ib1, gs0, gs1, zs) = refs
        gv, dv, rows = (g0, g1), (d0, d1), (r0, r1)
        glc, dlc, dlk = (gc0, gc1), (dc0, dc1), (dk0, dk1)
        ibs, gss = (ib0, ib1), (gs0, gs1)
        c = lax.axis_index("c")
        s = lax.axis_index("s")
        lo = c * half
        iota16 = lax.iota(jnp.int32, 16)

        def fire_iblock(bbv, par):
            base = s * eps + bbv * ib
            pltpu.async_copy(g_hbm.at[pl.ds(base, ib)], gv[par], ibs[par])
            pltpu.async_copy(s_hbm.at[pl.ds(base, ib)], dv[par], ibs[par])

        def wait_iblock(par):
            pltpu.make_async_copy(g_hbm.at[pl.ds(0, ib)], gv[par], ibs[par]).wait()
            pltpu.make_async_copy(s_hbm.at[pl.ds(0, ib)], dv[par], ibs[par]).wait()

        def compact(par):
            for i in range(ib // 16):
                glc[par][pl.ds(16 * i, 16)] = (16 * i) % 512 + iota16
                dlc[par][pl.ds(16 * i, 16)] = (half + (16 * i) % ntrash) + iota16
            offv = jnp.zeros((16,), jnp.int32)
            for i in range(ib // 16):
                d = dv[par][pl.ds(16 * i, 16)]
                g = gv[par][pl.ds(16 * i, 16)]
                mm = (d >= lo) & (d < lo + half)
                mmi = mm.astype(jnp.int32)
                cs = mmi
                for st in (1, 2, 4, 8):
                    sh = cs.at[jnp.maximum(iota16 - st, 0)].get(
                        mode="promise_in_bounds")
                    cs = cs + jnp.where(iota16 >= st, sh, 0)
                pos = jnp.where(mm, offv + cs - 1, ib + iota16)
                plsc.store_scatter(glc[par], [pos], g)
                plsc.store_scatter(dlc[par], [pos], d - lo)
                offv = offv + plsc.all_reduce_population_count(mm)
            cntb[...] = offv
            return cntb[...][0]

        def fire_g(par, ipar, cc):
            pltpu.async_copy(tbl_hbm.at[glc[ipar].at[pl.ds(cc * k, k)]],
                             rows[par], gss[par])

        def wait_g(par):
            pltpu.make_async_copy(tbl_hbm.at[glc[0].at[pl.ds(0, k)]],
                                  rows[par], gss[par]).wait()

        def sync_s(par, ipar, cc):
            for i in range(k // 16):
                dlk[par][pl.ds(16 * i, 16)] = dlc[ipar][pl.ds(cc * k + 16 * i, 16)]
            pltpu.sync_copy(rows[par], acc.at[dlk[par]], add=True)

        fire_iblock(0, 0)
        fire_iblock(1, 1)
        z16 = jnp.zeros((16,), jnp.float32)
        for r in range(ZR):
            for j in range(width // 16):
                zbuf[r, pl.ds(16 * j, 16)] = z16

        def zfire(t, carry):
            kk = s + NS * t

            @pl.when(kk < copies)
            def _():
                pltpu.async_copy(zbuf, acc.at[pl.ds(kk * ZR, ZR)], zs)
            return carry

        def zdrain(t, carry):
            kk = s + NS * t

            @pl.when(kk < copies)
            def _():
                pltpu.make_async_copy(zbuf, acc.at[pl.ds(0, ZR)], zs).wait()
            return carry

        lax.fori_loop(0, zt, zfire, None)
        lax.fori_loop(0, zt, zdrain, None)
        plsc.subcore_barrier()
        wait_iblock(0)
        n0 = compact(0)

        def block_pair(bb2, n_cur):
            for b in (0, 1):
                bb = 2 * bb2 + b
                nch2 = (n_cur + 2 * k - 1) // (2 * k)

                @pl.when(nch2 > 0)
                def _(b=b):
                    fire_g(0, b, 0)

                wait_iblock(1 - b)
                n_cur = compact(1 - b)
                fire_iblock(jnp.minimum(bb + 2, nb - 1), b)

                def inner(t, carry, b=b, nch2=nch2):
                    for b2 in (0, 1):
                        cc = 2 * t + b2
                        wait_g(b2)

                        @pl.when(cc + 1 < 2 * nch2)
                        def _(b=b, b2=b2, cc=cc):
                            fire_g(1 - b2, b, cc + 1)
                        sync_s(b2, b, cc)
                    return carry

                lax.fori_loop(0, nch2, inner, None)
            return n_cur

        lax.fori_loop(0, nb // 2, block_pair, n0)

        if with_tgt:
            w = s * 2 + c
            for q in range(NTGT // (2 * NS * KT)):
                tb = w * (NTGT // (2 * NS)) + q * KT
                pltpu.sync_copy(tidx_hbm.at[pl.ds(tb, KT)], ti_v)
                pltpu.async_copy(enc_hbm.at[ti_v], tr_v, gs0).wait()
                pltpu.sync_copy(tr_v, tout_hbm.at[pl.ds(tb, KT)])

        plsc.subcore_barrier()

        @pl.when(s < NS - 1)
        def _():
            pltpu.sync_copy(acc.at[pl.ds(s * sp, sp)],
                            out_hbm.at[pl.ds(c * half + s * sp, sp)])

        @pl.when(s == NS - 1)
        def _():
            pltpu.sync_copy(acc.at[pl.ds((NS - 1) * sp, sp_last)],
                            out_hbm.at[pl.ds(c * half + (NS - 1) * sp, sp_last)])

    return functools.partial(
        pl.kernel, mesh=_MESH,
        compiler_params=pltpu.CompilerParams(use_tc_tiling_on_sc=False, needs_layout_passes=False),
        out_type=out_type if with_tgt else out_type[0],
        scratch_types=scratch,
    )(body)


EOP = 819200

_obs_kernel = _make_agg_kernel(D, N // 2, 128, 8, 50, False)
_ins_kernel = _make_agg_kernel(TD, NU // 2, 112, 14, 4, True)


def _obs_segsum(y, obs_src, obs_dst):
    npad = EOP - EO
    pad_g = jnp.arange(npad, dtype=jnp.int32) % N
    pad_s = jnp.full((npad,), N, jnp.int32)
    return _obs_kernel(y, jnp.concatenate([obs_src, pad_g]),
                       jnp.concatenate([obs_dst, pad_s]))


def _sparse_stage4(t_iu, t_ui, ins_u, ins_i, tgtu_all, tgti_all, xu_enc, xi_enc):
    npad = EIP - EI
    pad_g = (jnp.arange(npad, dtype=jnp.int32) % NU)
    pad_s = jnp.full((npad,), NU, jnp.int32)
    g0 = jnp.concatenate([ins_i, pad_g])
    s0 = jnp.concatenate([ins_u, pad_s])
    g1 = jnp.concatenate([ins_u, pad_g])
    s1 = jnp.concatenate([ins_i, pad_s])
    agg_iu, xu_tgt = _ins_kernel(t_iu, g0, s0, xu_enc, tgtu_all)
    agg_ui, xi_tgt = _ins_kernel(t_ui, g1, s1, xi_enc, tgti_all)
    return agg_iu, agg_ui, xu_tgt, xi_tgt


def kernel(ts_diff, obs_src, obs_dst, ins_u, ins_i, tgt_u, tgt_i, tgt_u_neg,
           tgt_i_neg, xu_in, xi_in, embeds_u, embeds_i, W_cg, W_uu, b_uu, W_ii,
           b_ii, W_ui, W_iu, W_up, b_up, W_ip, b_ip):
    x_t = jnp.concatenate([xu_in, xi_in], axis=0)
    y, pmax = _stage1(x_t, W_cg)
    norm2 = jnp.max(pmax).reshape(1, 1)
    tsd = ts_diff.reshape(1, 1)

    msg = _obs_segsum(y, obs_src, obs_dst)

    xu_enc, t_ui, xu_lin = _stage3(xu_in, embeds_u, msg, 0, W_ui, W_uu, norm2, tsd)
    xi_enc, t_iu, xi_lin = _stage3(xi_in, embeds_i, msg, NU, W_iu, W_ii, norm2, tsd)

    tgtu_all = jnp.concatenate([tgt_u.reshape(-1), tgt_u_neg.reshape(-1)])
    tgti_all = jnp.concatenate([tgt_i.reshape(-1), tgt_i_neg.reshape(-1)])
    agg_iu, agg_ui, xu_tgt, xi_tgt = _sparse_stage4(
        t_iu, t_ui, ins_u, ins_i, tgtu_all, tgti_all, xu_enc, xi_enc)

    xu_tp, part_u = _stage5(xu_lin, agg_iu, xu_enc, b_uu)
    xi_tp, part_i = _stage5(xi_lin, agg_ui, xi_enc, b_ii)

    lr, lj = _predictor(xu_tgt, xi_tgt, W_up, b_up, W_ip, b_ip, part_u, part_i)

    B = tgt_u.shape[0]
    loss_rec = lr.reshape(())
    loss_jump = lj.reshape(())
    xu_pos = xu_tgt[:B].reshape(B, 1, 2 * D)
    xi_enc_out = xi_enc.reshape(NI, 1, 2 * D)
    return (loss_rec, loss_jump, xu_tp, xi_tp, xu_pos, xi_enc_out)
